# Initial kernel scaffold; baseline (speedup 1.0000x reference)
import jax
import jax.numpy as jnp
from jax.experimental import pallas as pl


def kernel(x, edge_index, W1, a_src1, a_dst1, b1, W2, a_src2, a_dst2, b2, W3, a_src3, a_dst3, b3):
    raise NotImplementedError("write your pallas kernel here")


# speedup vs baseline: 40.9397x; 40.9397x over previous
import functools
import jax
import jax.numpy as jnp
from jax import lax
from jax.experimental import pallas as pl
from jax.experimental.pallas import tpu as pltpu
from jax.experimental.pallas import tpu_sc as plsc

N = 10000
NP = 10240
DN = N
E = 320000
CHUNK = 128
NCHB = 162
NCHS = 81
EP = 16 * NCHB * CHUNK
ROWB = 512
GRID = NP // ROWB
EPS = 1e-16


def _leaky01(x):
    return jnp.where(x >= 0, x, 0.01 * x)


def _split_outs(h, asv, adv, glo, ghi, h2_ref, sas2_ref, sad2_ref):
    h2_ref[0] = h[:, :64]
    h2_ref[1] = h[:, 64:]
    hs = h * asv
    hd = h * adv
    sas2_ref[0] = jnp.dot(hs, glo, preferred_element_type=jnp.float32)
    sas2_ref[1] = jnp.dot(hs, ghi, preferred_element_type=jnp.float32)
    sad2_ref[0] = jnp.dot(hd, glo, preferred_element_type=jnp.float32)
    sad2_ref[1] = jnp.dot(hd, ghi, preferred_element_type=jnp.float32)


def _tc_first_body(x_ref, w_ref, asv_ref, adv_ref, glo_ref, ghi_ref,
                   h2_ref, sas2_ref, sad2_ref):
    h = jnp.dot(x_ref[...], w_ref[...], preferred_element_type=jnp.float32)
    _split_outs(h, asv_ref[...], adv_ref[...], glo_ref[...], ghi_ref[...],
                h2_ref, sas2_ref, sad2_ref)


def _tc_first(xp, w, asv, adv, glo, ghi):
    return pl.pallas_call(
        _tc_first_body,
        grid=(GRID,),
        in_specs=[
            pl.BlockSpec((ROWB, 128), lambda i: (i, 0)),
            pl.BlockSpec((128, 128), lambda i: (0, 0)),
            pl.BlockSpec((1, 128), lambda i: (0, 0)),
            pl.BlockSpec((1, 128), lambda i: (0, 0)),
            pl.BlockSpec((128, 16), lambda i: (0, 0)),
            pl.BlockSpec((128, 16), lambda i: (0, 0)),
        ],
        out_specs=[
            pl.BlockSpec((2, ROWB, 64), lambda i: (0, i, 0)),
            pl.BlockSpec((2, ROWB, 16), lambda i: (0, i, 0)),
            pl.BlockSpec((2, ROWB, 16), lambda i: (0, i, 0)),
        ],
        out_shape=[
            jax.ShapeDtypeStruct((2, NP, 64), jnp.float32),
            jax.ShapeDtypeStruct((2, NP, 16), jnp.float32),
            jax.ShapeDtypeStruct((2, NP, 16), jnp.float32),
        ],
    )(xp, w, asv, adv, glo, ghi)


def _combine_x(accm_ref, acce_ref, blo_ref, bhi_ref, gta_ref):
    gta = gta_ref[...]
    d0 = jnp.dot(acce_ref[0], gta, preferred_element_type=jnp.float32)
    d1 = jnp.dot(acce_ref[1], gta, preferred_element_type=jnp.float32)
    x0 = _leaky01(accm_ref[0] / (d0 + EPS) + blo_ref[...])
    x1 = _leaky01(accm_ref[1] / (d1 + EPS) + bhi_ref[...])
    return x0, x1


def _tc_mid_body(accm_ref, acce_ref, blo_ref, bhi_ref, gta_ref, wl_ref, wh_ref,
                 asv_ref, adv_ref, glo_ref, ghi_ref, h2_ref, sas2_ref, sad2_ref):
    x0, x1 = _combine_x(accm_ref, acce_ref, blo_ref, bhi_ref, gta_ref)
    h = (jnp.dot(x0, wl_ref[...], preferred_element_type=jnp.float32)
         + jnp.dot(x1, wh_ref[...], preferred_element_type=jnp.float32))
    _split_outs(h, asv_ref[...], adv_ref[...], glo_ref[...], ghi_ref[...],
                h2_ref, sas2_ref, sad2_ref)


def _tc_mid(accm, acce, blo, bhi, gta, wl, wh, asv, adv, glo, ghi):
    return pl.pallas_call(
        _tc_mid_body,
        grid=(GRID,),
        in_specs=[
            pl.BlockSpec((2, ROWB, 64), lambda i: (0, i, 0)),
            pl.BlockSpec((2, ROWB, 16), lambda i: (0, i, 0)),
            pl.BlockSpec((1, 64), lambda i: (0, 0)),
            pl.BlockSpec((1, 64), lambda i: (0, 0)),
            pl.BlockSpec((16, 64), lambda i: (0, 0)),
            pl.BlockSpec((64, 128), lambda i: (0, 0)),
            pl.BlockSpec((64, 128), lambda i: (0, 0)),
            pl.BlockSpec((1, 128), lambda i: (0, 0)),
            pl.BlockSpec((1, 128), lambda i: (0, 0)),
            pl.BlockSpec((128, 16), lambda i: (0, 0)),
            pl.BlockSpec((128, 16), lambda i: (0, 0)),
        ],
        out_specs=[
            pl.BlockSpec((2, ROWB, 64), lambda i: (0, i, 0)),
            pl.BlockSpec((2, ROWB, 16), lambda i: (0, i, 0)),
            pl.BlockSpec((2, ROWB, 16), lambda i: (0, i, 0)),
        ],
        out_shape=[
            jax.ShapeDtypeStruct((2, NP, 64), jnp.float32),
            jax.ShapeDtypeStruct((2, NP, 16), jnp.float32),
            jax.ShapeDtypeStruct((2, NP, 16), jnp.float32),
        ],
    )(accm, acce, blo, bhi, gta, wl, wh, asv, adv, glo, ghi)


def _tc_mid3_body(accm_ref, acce_ref, blo_ref, bhi_ref, gta_ref, w3l_ref, w3h_ref,
                  asv3_ref, adv3_ref, ones_ref, c2_ref, h3_ref, sas3_ref, sad3_ref):
    x0, x1 = _combine_x(accm_ref, acce_ref, blo_ref, bhi_ref, gta_ref)
    h3 = (jnp.dot(x0, w3l_ref[...], preferred_element_type=jnp.float32)
          + jnp.dot(x1, w3h_ref[...], preferred_element_type=jnp.float32))
    ones = ones_ref[...]
    sas3_ref[...] = jnp.dot(h3 * asv3_ref[...], ones, preferred_element_type=jnp.float32)
    sad3_ref[...] = jnp.dot(h3 * adv3_ref[...], ones, preferred_element_type=jnp.float32)
    h3_ref[...] = h3 + c2_ref[...]


def _tc_mid3(accm, acce, blo, bhi, gta, w3l, w3h, asv3, adv3, ones16, c2row):
    return pl.pallas_call(
        _tc_mid3_body,
        grid=(GRID,),
        in_specs=[
            pl.BlockSpec((2, ROWB, 64), lambda i: (0, i, 0)),
            pl.BlockSpec((2, ROWB, 16), lambda i: (0, i, 0)),
            pl.BlockSpec((1, 64), lambda i: (0, 0)),
            pl.BlockSpec((1, 64), lambda i: (0, 0)),
            pl.BlockSpec((16, 64), lambda i: (0, 0)),
            pl.BlockSpec((64, 16), lambda i: (0, 0)),
            pl.BlockSpec((64, 16), lambda i: (0, 0)),
            pl.BlockSpec((1, 16), lambda i: (0, 0)),
            pl.BlockSpec((1, 16), lambda i: (0, 0)),
            pl.BlockSpec((16, 16), lambda i: (0, 0)),
            pl.BlockSpec((1, 16), lambda i: (0, 0)),
        ],
        out_specs=[
            pl.BlockSpec((ROWB, 16), lambda i: (i, 0)),
            pl.BlockSpec((ROWB, 16), lambda i: (i, 0)),
            pl.BlockSpec((ROWB, 16), lambda i: (i, 0)),
        ],
        out_shape=[
            jax.ShapeDtypeStruct((NP, 16), jnp.float32),
            jax.ShapeDtypeStruct((NP, 16), jnp.float32),
            jax.ShapeDtypeStruct((NP, 16), jnp.float32),
        ],
    )(accm, acce, blo, bhi, gta, w3l, w3h, asv3, adv3, ones16, c2row)


def _tc_fin_body(acc_ref, b_ref, e2_ref, o_ref):
    r = acc_ref[0] + acc_ref[1]
    d = jnp.dot(r, e2_ref[...], preferred_element_type=jnp.float32)
    o_ref[...] = r / (d + EPS) + b_ref[...]


def _tc_fin(acc3, b3p, e2):
    return pl.pallas_call(
        _tc_fin_body,
        grid=(GRID,),
        in_specs=[
            pl.BlockSpec((2, ROWB, 16), lambda i: (0, i, 0)),
            pl.BlockSpec((1, 16), lambda i: (0, 0)),
            pl.BlockSpec((16, 16), lambda i: (0, 0)),
        ],
        out_specs=pl.BlockSpec((ROWB, 16), lambda i: (i, 0)),
        out_shape=jax.ShapeDtypeStruct((NP, 16), jnp.float32),
    )(acc3, b3p, e2)


_MESH = plsc.VectorSubcoreMesh(core_axis_name="c", subcore_axis_name="s")
_RPT = NP // 16


def _sc_big_body(h2_hbm, sas2_hbm, sad2_hbm, src_hbm, dst_hbm, outm_hbm, oute_hbm,
                 idx_s, idx_d, hg, sasg, sadg, mb, eb, accm, acce,
                 sem1, sem2, sem3):
    cid = lax.axis_index("c")
    sid = lax.axis_index("s")

    zv = jnp.zeros((16,), jnp.float32)

    def zero_row(r, carry):
        for j in range(4):
            mb[r, pl.ds(16 * j, 16)] = zv
        eb[r, pl.ds(0, 16)] = zv
        return carry

    lax.fori_loop(0, CHUNK, zero_row, 0)
    for j in range(_RPT // CHUNK):
        pltpu.sync_copy(mb, accm.at[pl.ds(sid * _RPT + j * CHUNK, CHUNK)])
        pltpu.sync_copy(eb, acce.at[pl.ds(sid * _RPT + j * CHUNK, CHUNK)])
    plsc.subcore_barrier()

    pltpu.sync_copy(src_hbm.at[sid], idx_s)
    pltpu.sync_copy(dst_hbm.at[sid], idx_d)

    def chunk_body(ci, carry):
        pltpu.async_copy(sas2_hbm.at[cid].at[idx_s.at[ci]], sasg, sem1)
        pltpu.async_copy(sad2_hbm.at[cid].at[idx_d.at[ci]], sadg, sem2)
        pltpu.async_copy(h2_hbm.at[cid].at[idx_s.at[ci]], hg, sem3)
        pltpu.make_async_copy(sas2_hbm.at[cid].at[idx_s.at[ci]], sasg, sem1).wait()
        pltpu.make_async_copy(sad2_hbm.at[cid].at[idx_d.at[ci]], sadg, sem2).wait()
        pltpu.make_async_copy(h2_hbm.at[cid].at[idx_s.at[ci]], hg, sem3).wait()

        def edge_body(e, c2):
            al = sasg[e, pl.ds(0, 16)] + sadg[e, pl.ds(0, 16)]
            al = jnp.where(al >= 0, al, al * 0.2)
            exv = jnp.exp(al)
            eb[e, pl.ds(0, 16)] = exv
            for k in range(4):
                mb[e, pl.ds(16 * k, 16)] = hg[e, pl.ds(16 * k, 16)] * exv[k]
            return c2

        lax.fori_loop(0, CHUNK, edge_body, 0)
        pltpu.sync_copy(mb, accm.at[idx_d.at[ci]], add=True)
        pltpu.sync_copy(eb, acce.at[idx_d.at[ci]], add=True)
        return carry

    lax.fori_loop(0, NCHB, chunk_body, 0)
    plsc.subcore_barrier()

    for j in range(_RPT // CHUNK):
        r0 = sid * _RPT + j * CHUNK
        pltpu.sync_copy(accm.at[pl.ds(r0, CHUNK)], hg)
        pltpu.sync_copy(hg, outm_hbm.at[cid, pl.ds(r0, CHUNK)])
        pltpu.sync_copy(acce.at[pl.ds(r0, CHUNK)], eb)
        pltpu.sync_copy(eb, oute_hbm.at[cid, pl.ds(r0, CHUNK)])


_sc_big = functools.partial(
    pl.kernel,
    out_type=[
        jax.ShapeDtypeStruct((2, NP, 64), jnp.float32),
        jax.ShapeDtypeStruct((2, NP, 16), jnp.float32),
    ],
    mesh=_MESH,
    compiler_params=pltpu.CompilerParams(use_tc_tiling_on_sc=False),
    scratch_types=[
        pltpu.VMEM((NCHB, CHUNK), jnp.int32),
        pltpu.VMEM((NCHB, CHUNK), jnp.int32),
        pltpu.VMEM((CHUNK, 64), jnp.float32),
        pltpu.VMEM((CHUNK, 16), jnp.float32),
        pltpu.VMEM((CHUNK, 16), jnp.float32),
        pltpu.VMEM((CHUNK, 64), jnp.float32),
        pltpu.VMEM((CHUNK, 16), jnp.float32),
        pltpu.VMEM_SHARED((NP, 64), jnp.float32),
        pltpu.VMEM_SHARED((NP, 16), jnp.float32),
        pltpu.SemaphoreType.DMA,
        pltpu.SemaphoreType.DMA,
        pltpu.SemaphoreType.DMA,
    ],
)(_sc_big_body)


def _sc_small_body(h3_hbm, sas3_hbm, sad3_hbm, src_hbm, dst_hbm, out_hbm,
                   idx_s, idx_d, hg, sasg, sadg, mb, acc,
                   sem1, sem2, sem3):
    cid = lax.axis_index("c")
    sid = lax.axis_index("s")
    wid = sid * 2 + cid

    zv = jnp.zeros((16,), jnp.float32)

    def zero_row(r, carry):
        mb[r, pl.ds(0, 16)] = zv
        return carry

    lax.fori_loop(0, CHUNK, zero_row, 0)
    for j in range(_RPT // CHUNK):
        pltpu.sync_copy(mb, acc.at[pl.ds(sid * _RPT + j * CHUNK, CHUNK)])
    plsc.subcore_barrier()

    pltpu.sync_copy(src_hbm.at[wid], idx_s)
    pltpu.sync_copy(dst_hbm.at[wid], idx_d)

    def chunk_body(ci, carry):
        pltpu.async_copy(sas3_hbm.at[idx_s.at[ci]], sasg, sem1)
        pltpu.async_copy(sad3_hbm.at[idx_d.at[ci]], sadg, sem2)
        pltpu.async_copy(h3_hbm.at[idx_s.at[ci]], hg, sem3)
        pltpu.make_async_copy(sas3_hbm.at[idx_s.at[ci]], sasg, sem1).wait()
        pltpu.make_async_copy(sad3_hbm.at[idx_d.at[ci]], sadg, sem2).wait()
        pltpu.make_async_copy(h3_hbm.at[idx_s.at[ci]], hg, sem3).wait()

        def edge_body(e, c2):
            al = sasg[e, pl.ds(0, 16)] + sadg[e, pl.ds(0, 16)]
            al = jnp.where(al >= 0, al, al * 0.2)
            exv = jnp.exp(al)
            mb[e, pl.ds(0, 16)] = hg[e, pl.ds(0, 16)] * exv
            return c2

        lax.fori_loop(0, CHUNK, edge_body, 0)
        pltpu.sync_copy(mb, acc.at[idx_d.at[ci]], add=True)
        return carry

    lax.fori_loop(0, NCHS, chunk_body, 0)
    plsc.subcore_barrier()

    for j in range(_RPT // CHUNK):
        r0 = sid * _RPT + j * CHUNK
        pltpu.sync_copy(acc.at[pl.ds(r0, CHUNK)], mb)
        pltpu.sync_copy(mb, out_hbm.at[cid, pl.ds(r0, CHUNK)])


_sc_small = functools.partial(
    pl.kernel,
    out_type=[jax.ShapeDtypeStruct((2, NP, 16), jnp.float32)],
    mesh=_MESH,
    compiler_params=pltpu.CompilerParams(use_tc_tiling_on_sc=False),
    scratch_types=[
        pltpu.VMEM((NCHS, CHUNK), jnp.int32),
        pltpu.VMEM((NCHS, CHUNK), jnp.int32),
        pltpu.VMEM((CHUNK, 16), jnp.float32),
        pltpu.VMEM((CHUNK, 16), jnp.float32),
        pltpu.VMEM((CHUNK, 16), jnp.float32),
        pltpu.VMEM((CHUNK, 16), jnp.float32),
        pltpu.VMEM_SHARED((NP, 16), jnp.float32),
        pltpu.SemaphoreType.DMA,
        pltpu.SemaphoreType.DMA,
        pltpu.SemaphoreType.DMA,
    ],
)(_sc_small_body)


def kernel(x, edge_index, W1, a_src1, a_dst1, b1, W2, a_src2, a_dst2, b2,
           W3, a_src3, a_dst3, b3):
    f32 = jnp.float32
    head = jnp.arange(128) // 16
    j16 = jnp.arange(16)
    glo = ((head[:, None] == j16[None, :] % 4) & (head[:, None] < 4)).astype(f32)
    ghi = ((head[:, None] - 4 == j16[None, :] % 4) & (head[:, None] >= 4)).astype(f32)
    h64 = jnp.arange(64) // 16
    gta = ((j16[:, None] == h64[None, :]) & (j16[:, None] < 4)).astype(f32)
    ones16 = jnp.ones((16, 16), f32)
    e2 = (j16[:, None] == 2).astype(f32) * jnp.ones((1, 16), f32)
    c2row = (j16[None, :] == 2).astype(f32)

    loop = jnp.arange(N, dtype=jnp.int32)
    pad = jnp.full((EP - E - N,), DN, jnp.int32)
    src_flat = jnp.concatenate([edge_index[0], loop, pad])
    dst_flat = jnp.concatenate([edge_index[1], loop, pad])
    src16 = src_flat.reshape(16, NCHB, CHUNK)
    dst16 = dst_flat.reshape(16, NCHB, CHUNK)
    src32 = src_flat.reshape(32, NCHS, CHUNK)
    dst32 = dst_flat.reshape(32, NCHS, CHUNK)

    xp = jnp.pad(x, ((0, NP - N), (0, 0)))
    asv1 = a_src1.reshape(1, 128)
    adv1 = a_dst1.reshape(1, 128)
    asv2 = a_src2.reshape(1, 128)
    adv2 = a_dst2.reshape(1, 128)
    w3p = jnp.pad(W3, ((0, 0), (0, 14)))
    asv3 = jnp.pad(a_src3.reshape(1, 2), ((0, 0), (0, 14)))
    adv3 = jnp.pad(a_dst3.reshape(1, 2), ((0, 0), (0, 14)))
    b1lo = b1[:64].reshape(1, 64)
    b1hi = b1[64:].reshape(1, 64)
    b2lo = b2[:64].reshape(1, 64)
    b2hi = b2[64:].reshape(1, 64)
    b3p = jnp.pad(b3.reshape(1, 2), ((0, 0), (0, 14)))

    h2a, sas, sad = _tc_first(xp, W1, asv1, adv1, glo, ghi)
    accm, acce = _sc_big(h2a, sas, sad, src16, dst16)
    h2b, sas, sad = _tc_mid(accm, acce, b1lo, b1hi, gta,
                            W2[:64], W2[64:], asv2, adv2, glo, ghi)
    accm, acce = _sc_big(h2b, sas, sad, src16, dst16)
    h3t, sas3, sad3 = _tc_mid3(accm, acce, b2lo, b2hi, gta,
                               w3p[:64], w3p[64:], asv3, adv3, ones16, c2row)
    (acc3,) = _sc_small(h3t, sas3, sad3, src32, dst32)
    o = _tc_fin(acc3, b3p, e2)
    return o[:N, :2]


# speedup vs baseline: 51.2509x; 1.2519x over previous
import functools
import jax
import jax.numpy as jnp
from jax import lax
from jax.experimental import pallas as pl
from jax.experimental.pallas import tpu as pltpu
from jax.experimental.pallas import tpu_sc as plsc

N = 10000
NP = 10240
DN = N
E = 320000
CHUNK = 128
NCHB = 162
NCHS = 81
EP = 16 * NCHB * CHUNK
ROWB = 512
GRID = NP // ROWB
EPS = 1e-16


def _leaky01(x):
    return jnp.where(x >= 0, x, 0.01 * x)


def _split_outs(h, asv, adv, glo, ghi, h2_ref, sas2_ref, sad2_ref):
    h2_ref[0] = h[:, :64]
    h2_ref[1] = h[:, 64:]
    hs = h * asv
    hd = h * adv
    sas2_ref[0] = jnp.dot(hs, glo, preferred_element_type=jnp.float32)
    sas2_ref[1] = jnp.dot(hs, ghi, preferred_element_type=jnp.float32)
    sad2_ref[0] = jnp.dot(hd, glo, preferred_element_type=jnp.float32)
    sad2_ref[1] = jnp.dot(hd, ghi, preferred_element_type=jnp.float32)


def _tc_first_body(x_ref, w_ref, asv_ref, adv_ref, glo_ref, ghi_ref,
                   h2_ref, sas2_ref, sad2_ref):
    h = jnp.dot(x_ref[...], w_ref[...], preferred_element_type=jnp.float32)
    _split_outs(h, asv_ref[...], adv_ref[...], glo_ref[...], ghi_ref[...],
                h2_ref, sas2_ref, sad2_ref)


def _tc_first(xp, w, asv, adv, glo, ghi):
    return pl.pallas_call(
        _tc_first_body,
        grid=(GRID,),
        in_specs=[
            pl.BlockSpec((ROWB, 128), lambda i: (i, 0)),
            pl.BlockSpec((128, 128), lambda i: (0, 0)),
            pl.BlockSpec((1, 128), lambda i: (0, 0)),
            pl.BlockSpec((1, 128), lambda i: (0, 0)),
            pl.BlockSpec((128, 16), lambda i: (0, 0)),
            pl.BlockSpec((128, 16), lambda i: (0, 0)),
        ],
        out_specs=[
            pl.BlockSpec((2, ROWB, 64), lambda i: (0, i, 0)),
            pl.BlockSpec((2, ROWB, 16), lambda i: (0, i, 0)),
            pl.BlockSpec((2, ROWB, 16), lambda i: (0, i, 0)),
        ],
        out_shape=[
            jax.ShapeDtypeStruct((2, NP, 64), jnp.float32),
            jax.ShapeDtypeStruct((2, NP, 16), jnp.float32),
            jax.ShapeDtypeStruct((2, NP, 16), jnp.float32),
        ],
    )(xp, w, asv, adv, glo, ghi)


def _combine_x(accm_ref, acce_ref, blo_ref, bhi_ref, gta_ref):
    gta = gta_ref[...]
    d0 = jnp.dot(acce_ref[0], gta, preferred_element_type=jnp.float32)
    d1 = jnp.dot(acce_ref[1], gta, preferred_element_type=jnp.float32)
    x0 = _leaky01(accm_ref[0] / (d0 + EPS) + blo_ref[...])
    x1 = _leaky01(accm_ref[1] / (d1 + EPS) + bhi_ref[...])
    return x0, x1


def _tc_mid_body(accm_ref, acce_ref, blo_ref, bhi_ref, gta_ref, wl_ref, wh_ref,
                 asv_ref, adv_ref, glo_ref, ghi_ref, h2_ref, sas2_ref, sad2_ref):
    x0, x1 = _combine_x(accm_ref, acce_ref, blo_ref, bhi_ref, gta_ref)
    h = (jnp.dot(x0, wl_ref[...], preferred_element_type=jnp.float32)
         + jnp.dot(x1, wh_ref[...], preferred_element_type=jnp.float32))
    _split_outs(h, asv_ref[...], adv_ref[...], glo_ref[...], ghi_ref[...],
                h2_ref, sas2_ref, sad2_ref)


def _tc_mid(accm, acce, blo, bhi, gta, wl, wh, asv, adv, glo, ghi):
    return pl.pallas_call(
        _tc_mid_body,
        grid=(GRID,),
        in_specs=[
            pl.BlockSpec((2, ROWB, 64), lambda i: (0, i, 0)),
            pl.BlockSpec((2, ROWB, 16), lambda i: (0, i, 0)),
            pl.BlockSpec((1, 64), lambda i: (0, 0)),
            pl.BlockSpec((1, 64), lambda i: (0, 0)),
            pl.BlockSpec((16, 64), lambda i: (0, 0)),
            pl.BlockSpec((64, 128), lambda i: (0, 0)),
            pl.BlockSpec((64, 128), lambda i: (0, 0)),
            pl.BlockSpec((1, 128), lambda i: (0, 0)),
            pl.BlockSpec((1, 128), lambda i: (0, 0)),
            pl.BlockSpec((128, 16), lambda i: (0, 0)),
            pl.BlockSpec((128, 16), lambda i: (0, 0)),
        ],
        out_specs=[
            pl.BlockSpec((2, ROWB, 64), lambda i: (0, i, 0)),
            pl.BlockSpec((2, ROWB, 16), lambda i: (0, i, 0)),
            pl.BlockSpec((2, ROWB, 16), lambda i: (0, i, 0)),
        ],
        out_shape=[
            jax.ShapeDtypeStruct((2, NP, 64), jnp.float32),
            jax.ShapeDtypeStruct((2, NP, 16), jnp.float32),
            jax.ShapeDtypeStruct((2, NP, 16), jnp.float32),
        ],
    )(accm, acce, blo, bhi, gta, wl, wh, asv, adv, glo, ghi)


def _tc_mid3_body(accm_ref, acce_ref, blo_ref, bhi_ref, gta_ref, w3l_ref, w3h_ref,
                  asv3_ref, adv3_ref, ones_ref, c2_ref, h3_ref, sas3_ref, sad3_ref):
    x0, x1 = _combine_x(accm_ref, acce_ref, blo_ref, bhi_ref, gta_ref)
    h3 = (jnp.dot(x0, w3l_ref[...], preferred_element_type=jnp.float32)
          + jnp.dot(x1, w3h_ref[...], preferred_element_type=jnp.float32))
    ones = ones_ref[...]
    sas3_ref[...] = jnp.dot(h3 * asv3_ref[...], ones, preferred_element_type=jnp.float32)
    sad3_ref[...] = jnp.dot(h3 * adv3_ref[...], ones, preferred_element_type=jnp.float32)
    h3_ref[...] = h3 + c2_ref[...]


def _tc_mid3(accm, acce, blo, bhi, gta, w3l, w3h, asv3, adv3, ones16, c2row):
    return pl.pallas_call(
        _tc_mid3_body,
        grid=(GRID,),
        in_specs=[
            pl.BlockSpec((2, ROWB, 64), lambda i: (0, i, 0)),
            pl.BlockSpec((2, ROWB, 16), lambda i: (0, i, 0)),
            pl.BlockSpec((1, 64), lambda i: (0, 0)),
            pl.BlockSpec((1, 64), lambda i: (0, 0)),
            pl.BlockSpec((16, 64), lambda i: (0, 0)),
            pl.BlockSpec((64, 16), lambda i: (0, 0)),
            pl.BlockSpec((64, 16), lambda i: (0, 0)),
            pl.BlockSpec((1, 16), lambda i: (0, 0)),
            pl.BlockSpec((1, 16), lambda i: (0, 0)),
            pl.BlockSpec((16, 16), lambda i: (0, 0)),
            pl.BlockSpec((1, 16), lambda i: (0, 0)),
        ],
        out_specs=[
            pl.BlockSpec((ROWB, 16), lambda i: (i, 0)),
            pl.BlockSpec((ROWB, 16), lambda i: (i, 0)),
            pl.BlockSpec((ROWB, 16), lambda i: (i, 0)),
        ],
        out_shape=[
            jax.ShapeDtypeStruct((NP, 16), jnp.float32),
            jax.ShapeDtypeStruct((NP, 16), jnp.float32),
            jax.ShapeDtypeStruct((NP, 16), jnp.float32),
        ],
    )(accm, acce, blo, bhi, gta, w3l, w3h, asv3, adv3, ones16, c2row)


def _tc_fin_body(acc_ref, b_ref, e2_ref, o_ref):
    r = acc_ref[0] + acc_ref[1]
    d = jnp.dot(r, e2_ref[...], preferred_element_type=jnp.float32)
    o_ref[...] = r / (d + EPS) + b_ref[...]


def _tc_fin(acc3, b3p, e2):
    return pl.pallas_call(
        _tc_fin_body,
        grid=(GRID,),
        in_specs=[
            pl.BlockSpec((2, ROWB, 16), lambda i: (0, i, 0)),
            pl.BlockSpec((1, 16), lambda i: (0, 0)),
            pl.BlockSpec((16, 16), lambda i: (0, 0)),
        ],
        out_specs=pl.BlockSpec((ROWB, 16), lambda i: (i, 0)),
        out_shape=jax.ShapeDtypeStruct((NP, 16), jnp.float32),
    )(acc3, b3p, e2)


_MESH = plsc.VectorSubcoreMesh(core_axis_name="c", subcore_axis_name="s")
_RPT = NP // 16


def _sc_big_body(h2_hbm, sas2_hbm, sad2_hbm, src_hbm, dst_hbm, outm_hbm, oute_hbm,
                 idx_s, idx_d, hg0, hg1, sasg0, sasg1, sadg0, sadg1,
                 mb, eb, accm, acce, gsem0, gsem1):
    cid = lax.axis_index("c")
    sid = lax.axis_index("s")
    hg = (hg0, hg1)
    sasg = (sasg0, sasg1)
    sadg = (sadg0, sadg1)
    gsem = (gsem0, gsem1)

    zv = jnp.zeros((16,), jnp.float32)

    def zero_row(r, carry):
        for j in range(4):
            mb[r, pl.ds(16 * j, 16)] = zv
        eb[r, pl.ds(0, 16)] = zv
        return carry

    lax.fori_loop(0, CHUNK, zero_row, 0)
    for j in range(_RPT // CHUNK):
        pltpu.sync_copy(mb, accm.at[pl.ds(sid * _RPT + j * CHUNK, CHUNK)])
        pltpu.sync_copy(eb, acce.at[pl.ds(sid * _RPT + j * CHUNK, CHUNK)])
    plsc.subcore_barrier()

    pltpu.sync_copy(src_hbm.at[sid], idx_s)
    pltpu.sync_copy(dst_hbm.at[sid], idx_d)

    def fire_gather(cc, b):
        pltpu.async_copy(sas2_hbm.at[cid].at[idx_s.at[cc]], sasg[b], gsem[b])
        pltpu.async_copy(sad2_hbm.at[cid].at[idx_d.at[cc]], sadg[b], gsem[b])
        pltpu.async_copy(h2_hbm.at[cid].at[idx_s.at[cc]], hg[b], gsem[b])

    def drain_gather(cc, b):
        pltpu.make_async_copy(sas2_hbm.at[cid].at[idx_s.at[cc]], sasg[b], gsem[b]).wait()
        pltpu.make_async_copy(sad2_hbm.at[cid].at[idx_d.at[cc]], sadg[b], gsem[b]).wait()
        pltpu.make_async_copy(h2_hbm.at[cid].at[idx_s.at[cc]], hg[b], gsem[b]).wait()

    fire_gather(0, 0)
    fire_gather(1, 1)

    def body(i, carry):
        ci = 2 * i
        for b in range(2):
            cc = ci + b
            drain_gather(cc, b)

            def edge_body(e, c2):
                al = sasg[b][e, pl.ds(0, 16)] + sadg[b][e, pl.ds(0, 16)]
                al = jnp.where(al >= 0, al, al * 0.2)
                exv = jnp.exp(al)
                eb[e, pl.ds(0, 16)] = exv
                for k in range(4):
                    mb[e, pl.ds(16 * k, 16)] = hg[b][e, pl.ds(16 * k, 16)] * exv[k]
                return c2

            lax.fori_loop(0, CHUNK, edge_body, 0, unroll=4)

            @pl.when(cc + 2 < NCHB)
            def _():
                fire_gather(cc + 2, b)

            pltpu.sync_copy(mb, accm.at[idx_d.at[cc]], add=True)
            pltpu.sync_copy(eb, acce.at[idx_d.at[cc]], add=True)
        return carry

    lax.fori_loop(0, NCHB // 2, body, 0)
    plsc.subcore_barrier()

    for j in range(_RPT // CHUNK):
        r0 = sid * _RPT + j * CHUNK
        pltpu.sync_copy(accm.at[pl.ds(r0, CHUNK)], hg0)
        pltpu.sync_copy(hg0, outm_hbm.at[cid, pl.ds(r0, CHUNK)])
        pltpu.sync_copy(acce.at[pl.ds(r0, CHUNK)], eb)
        pltpu.sync_copy(eb, oute_hbm.at[cid, pl.ds(r0, CHUNK)])


_sc_big = functools.partial(
    pl.kernel,
    out_type=[
        jax.ShapeDtypeStruct((2, NP, 64), jnp.float32),
        jax.ShapeDtypeStruct((2, NP, 16), jnp.float32),
    ],
    mesh=_MESH,
    compiler_params=pltpu.CompilerParams(use_tc_tiling_on_sc=False),
    scratch_types=[
        pltpu.VMEM((NCHB, CHUNK), jnp.int32),
        pltpu.VMEM((NCHB, CHUNK), jnp.int32),
        pltpu.VMEM((CHUNK, 64), jnp.float32),
        pltpu.VMEM((CHUNK, 64), jnp.float32),
        pltpu.VMEM((CHUNK, 16), jnp.float32),
        pltpu.VMEM((CHUNK, 16), jnp.float32),
        pltpu.VMEM((CHUNK, 16), jnp.float32),
        pltpu.VMEM((CHUNK, 16), jnp.float32),
        pltpu.VMEM((CHUNK, 64), jnp.float32),
        pltpu.VMEM((CHUNK, 16), jnp.float32),
        pltpu.VMEM_SHARED((NP, 64), jnp.float32),
        pltpu.VMEM_SHARED((NP, 16), jnp.float32),
        pltpu.SemaphoreType.DMA,
        pltpu.SemaphoreType.DMA,
    ],
)(_sc_big_body)


def _sc_small_body(h3_hbm, sas3_hbm, sad3_hbm, src_hbm, dst_hbm, out_hbm,
                   idx_s, idx_d, hg, sasg, sadg, mb, acc,
                   sem1, sem2, sem3):
    cid = lax.axis_index("c")
    sid = lax.axis_index("s")
    wid = sid * 2 + cid

    zv = jnp.zeros((16,), jnp.float32)

    def zero_row(r, carry):
        mb[r, pl.ds(0, 16)] = zv
        return carry

    lax.fori_loop(0, CHUNK, zero_row, 0)
    for j in range(_RPT // CHUNK):
        pltpu.sync_copy(mb, acc.at[pl.ds(sid * _RPT + j * CHUNK, CHUNK)])
    plsc.subcore_barrier()

    pltpu.sync_copy(src_hbm.at[wid], idx_s)
    pltpu.sync_copy(dst_hbm.at[wid], idx_d)

    def chunk_body(ci, carry):
        pltpu.async_copy(sas3_hbm.at[idx_s.at[ci]], sasg, sem1)
        pltpu.async_copy(sad3_hbm.at[idx_d.at[ci]], sadg, sem2)
        pltpu.async_copy(h3_hbm.at[idx_s.at[ci]], hg, sem3)
        pltpu.make_async_copy(sas3_hbm.at[idx_s.at[ci]], sasg, sem1).wait()
        pltpu.make_async_copy(sad3_hbm.at[idx_d.at[ci]], sadg, sem2).wait()
        pltpu.make_async_copy(h3_hbm.at[idx_s.at[ci]], hg, sem3).wait()

        def edge_body(e, c2):
            al = sasg[e, pl.ds(0, 16)] + sadg[e, pl.ds(0, 16)]
            al = jnp.where(al >= 0, al, al * 0.2)
            exv = jnp.exp(al)
            mb[e, pl.ds(0, 16)] = hg[e, pl.ds(0, 16)] * exv
            return c2

        lax.fori_loop(0, CHUNK, edge_body, 0)
        pltpu.sync_copy(mb, acc.at[idx_d.at[ci]], add=True)
        return carry

    lax.fori_loop(0, NCHS, chunk_body, 0)
    plsc.subcore_barrier()

    for j in range(_RPT // CHUNK):
        r0 = sid * _RPT + j * CHUNK
        pltpu.sync_copy(acc.at[pl.ds(r0, CHUNK)], mb)
        pltpu.sync_copy(mb, out_hbm.at[cid, pl.ds(r0, CHUNK)])


_sc_small = functools.partial(
    pl.kernel,
    out_type=[jax.ShapeDtypeStruct((2, NP, 16), jnp.float32)],
    mesh=_MESH,
    compiler_params=pltpu.CompilerParams(use_tc_tiling_on_sc=False),
    scratch_types=[
        pltpu.VMEM((NCHS, CHUNK), jnp.int32),
        pltpu.VMEM((NCHS, CHUNK), jnp.int32),
        pltpu.VMEM((CHUNK, 16), jnp.float32),
        pltpu.VMEM((CHUNK, 16), jnp.float32),
        pltpu.VMEM((CHUNK, 16), jnp.float32),
        pltpu.VMEM((CHUNK, 16), jnp.float32),
        pltpu.VMEM_SHARED((NP, 16), jnp.float32),
        pltpu.SemaphoreType.DMA,
        pltpu.SemaphoreType.DMA,
        pltpu.SemaphoreType.DMA,
    ],
)(_sc_small_body)


def kernel(x, edge_index, W1, a_src1, a_dst1, b1, W2, a_src2, a_dst2, b2,
           W3, a_src3, a_dst3, b3):
    f32 = jnp.float32
    head = jnp.arange(128) // 16
    j16 = jnp.arange(16)
    glo = ((head[:, None] == j16[None, :] % 4) & (head[:, None] < 4)).astype(f32)
    ghi = ((head[:, None] - 4 == j16[None, :] % 4) & (head[:, None] >= 4)).astype(f32)
    h64 = jnp.arange(64) // 16
    gta = ((j16[:, None] == h64[None, :]) & (j16[:, None] < 4)).astype(f32)
    ones16 = jnp.ones((16, 16), f32)
    e2 = (j16[:, None] == 2).astype(f32) * jnp.ones((1, 16), f32)
    c2row = (j16[None, :] == 2).astype(f32)

    loop = jnp.arange(N, dtype=jnp.int32)
    pad = jnp.full((EP - E - N,), DN, jnp.int32)
    src_flat = jnp.concatenate([edge_index[0], loop, pad])
    dst_flat = jnp.concatenate([edge_index[1], loop, pad])
    src16 = src_flat.reshape(16, NCHB, CHUNK)
    dst16 = dst_flat.reshape(16, NCHB, CHUNK)
    src32 = src_flat.reshape(32, NCHS, CHUNK)
    dst32 = dst_flat.reshape(32, NCHS, CHUNK)

    xp = jnp.pad(x, ((0, NP - N), (0, 0)))
    asv1 = a_src1.reshape(1, 128)
    adv1 = a_dst1.reshape(1, 128)
    asv2 = a_src2.reshape(1, 128)
    adv2 = a_dst2.reshape(1, 128)
    w3p = jnp.pad(W3, ((0, 0), (0, 14)))
    asv3 = jnp.pad(a_src3.reshape(1, 2), ((0, 0), (0, 14)))
    adv3 = jnp.pad(a_dst3.reshape(1, 2), ((0, 0), (0, 14)))
    b1lo = b1[:64].reshape(1, 64)
    b1hi = b1[64:].reshape(1, 64)
    b2lo = b2[:64].reshape(1, 64)
    b2hi = b2[64:].reshape(1, 64)
    b3p = jnp.pad(b3.reshape(1, 2), ((0, 0), (0, 14)))

    h2a, sas, sad = _tc_first(xp, W1, asv1, adv1, glo, ghi)
    accm, acce = _sc_big(h2a, sas, sad, src16, dst16)
    h2b, sas, sad = _tc_mid(accm, acce, b1lo, b1hi, gta,
                            W2[:64], W2[64:], asv2, adv2, glo, ghi)
    accm, acce = _sc_big(h2b, sas, sad, src16, dst16)
    h3t, sas3, sad3 = _tc_mid3(accm, acce, b2lo, b2hi, gta,
                               w3p[:64], w3p[64:], asv3, adv3, ones16, c2row)
    (acc3,) = _sc_small(h3t, sas3, sad3, src32, dst32)
    o = _tc_fin(acc3, b3p, e2)
    return o[:N, :2]


# speedup vs baseline: 121.6381x; 2.3734x over previous
import functools
import jax
import jax.numpy as jnp
from jax import lax
from jax.experimental import pallas as pl
from jax.experimental.pallas import tpu as pltpu
from jax.experimental.pallas import tpu_sc as plsc

N = 10000
NP = 10240
DN = N
E = 320000
CHUNK = 128
NCHB = 162
NCHS = 81
EP = 16 * NCHB * CHUNK
ROWB = 512
GRID = NP // ROWB
EPS = 1e-16


def _leaky01(x):
    return jnp.where(x >= 0, x, 0.01 * x)


def _split_outs(h, asv, adv, glo, ghi, h2_ref, sas2_ref, sad2_ref):
    h2_ref[0] = h[:, :64]
    h2_ref[1] = h[:, 64:]
    hs = h * asv
    hd = h * adv
    sas2_ref[0] = jnp.dot(hs, glo, preferred_element_type=jnp.float32)
    sas2_ref[1] = jnp.dot(hs, ghi, preferred_element_type=jnp.float32)
    sad2_ref[0] = jnp.dot(hd, glo, preferred_element_type=jnp.float32)
    sad2_ref[1] = jnp.dot(hd, ghi, preferred_element_type=jnp.float32)


def _tc_first_body(x_ref, w_ref, asv_ref, adv_ref, glo_ref, ghi_ref,
                   h2_ref, sas2_ref, sad2_ref):
    h = jnp.dot(x_ref[...], w_ref[...], preferred_element_type=jnp.float32)
    _split_outs(h, asv_ref[...], adv_ref[...], glo_ref[...], ghi_ref[...],
                h2_ref, sas2_ref, sad2_ref)


def _tc_first(xp, w, asv, adv, glo, ghi):
    return pl.pallas_call(
        _tc_first_body,
        grid=(GRID,),
        in_specs=[
            pl.BlockSpec((ROWB, 128), lambda i: (i, 0)),
            pl.BlockSpec((128, 128), lambda i: (0, 0)),
            pl.BlockSpec((1, 128), lambda i: (0, 0)),
            pl.BlockSpec((1, 128), lambda i: (0, 0)),
            pl.BlockSpec((128, 16), lambda i: (0, 0)),
            pl.BlockSpec((128, 16), lambda i: (0, 0)),
        ],
        out_specs=[
            pl.BlockSpec((2, ROWB, 64), lambda i: (0, i, 0)),
            pl.BlockSpec((2, ROWB, 16), lambda i: (0, i, 0)),
            pl.BlockSpec((2, ROWB, 16), lambda i: (0, i, 0)),
        ],
        out_shape=[
            jax.ShapeDtypeStruct((2, NP, 64), jnp.float32),
            jax.ShapeDtypeStruct((2, NP, 16), jnp.float32),
            jax.ShapeDtypeStruct((2, NP, 16), jnp.float32),
        ],
    )(xp, w, asv, adv, glo, ghi)


def _combine_x(accm_ref, acce_ref, blo_ref, bhi_ref, gta_ref):
    gta = gta_ref[...]
    d0 = jnp.dot(acce_ref[0], gta, preferred_element_type=jnp.float32)
    d1 = jnp.dot(acce_ref[1], gta, preferred_element_type=jnp.float32)
    x0 = _leaky01(accm_ref[0] / (d0 + EPS) + blo_ref[...])
    x1 = _leaky01(accm_ref[1] / (d1 + EPS) + bhi_ref[...])
    return x0, x1


def _tc_mid_body(accm_ref, acce_ref, blo_ref, bhi_ref, gta_ref, wl_ref, wh_ref,
                 asv_ref, adv_ref, glo_ref, ghi_ref, h2_ref, sas2_ref, sad2_ref):
    x0, x1 = _combine_x(accm_ref, acce_ref, blo_ref, bhi_ref, gta_ref)
    h = (jnp.dot(x0, wl_ref[...], preferred_element_type=jnp.float32)
         + jnp.dot(x1, wh_ref[...], preferred_element_type=jnp.float32))
    _split_outs(h, asv_ref[...], adv_ref[...], glo_ref[...], ghi_ref[...],
                h2_ref, sas2_ref, sad2_ref)


def _tc_mid(accm, acce, blo, bhi, gta, wl, wh, asv, adv, glo, ghi):
    return pl.pallas_call(
        _tc_mid_body,
        grid=(GRID,),
        in_specs=[
            pl.BlockSpec((2, ROWB, 64), lambda i: (0, i, 0)),
            pl.BlockSpec((2, ROWB, 16), lambda i: (0, i, 0)),
            pl.BlockSpec((1, 64), lambda i: (0, 0)),
            pl.BlockSpec((1, 64), lambda i: (0, 0)),
            pl.BlockSpec((16, 64), lambda i: (0, 0)),
            pl.BlockSpec((64, 128), lambda i: (0, 0)),
            pl.BlockSpec((64, 128), lambda i: (0, 0)),
            pl.BlockSpec((1, 128), lambda i: (0, 0)),
            pl.BlockSpec((1, 128), lambda i: (0, 0)),
            pl.BlockSpec((128, 16), lambda i: (0, 0)),
            pl.BlockSpec((128, 16), lambda i: (0, 0)),
        ],
        out_specs=[
            pl.BlockSpec((2, ROWB, 64), lambda i: (0, i, 0)),
            pl.BlockSpec((2, ROWB, 16), lambda i: (0, i, 0)),
            pl.BlockSpec((2, ROWB, 16), lambda i: (0, i, 0)),
        ],
        out_shape=[
            jax.ShapeDtypeStruct((2, NP, 64), jnp.float32),
            jax.ShapeDtypeStruct((2, NP, 16), jnp.float32),
            jax.ShapeDtypeStruct((2, NP, 16), jnp.float32),
        ],
    )(accm, acce, blo, bhi, gta, wl, wh, asv, adv, glo, ghi)


def _tc_mid3_body(accm_ref, acce_ref, blo_ref, bhi_ref, gta_ref, w3l_ref, w3h_ref,
                  asv3_ref, adv3_ref, ones_ref, c2_ref, h3_ref, sas3_ref, sad3_ref):
    x0, x1 = _combine_x(accm_ref, acce_ref, blo_ref, bhi_ref, gta_ref)
    h3 = (jnp.dot(x0, w3l_ref[...], preferred_element_type=jnp.float32)
          + jnp.dot(x1, w3h_ref[...], preferred_element_type=jnp.float32))
    ones = ones_ref[...]
    sas3_ref[...] = jnp.dot(h3 * asv3_ref[...], ones, preferred_element_type=jnp.float32)
    sad3_ref[...] = jnp.dot(h3 * adv3_ref[...], ones, preferred_element_type=jnp.float32)
    h3_ref[...] = h3 + c2_ref[...]


def _tc_mid3(accm, acce, blo, bhi, gta, w3l, w3h, asv3, adv3, ones16, c2row):
    return pl.pallas_call(
        _tc_mid3_body,
        grid=(GRID,),
        in_specs=[
            pl.BlockSpec((2, ROWB, 64), lambda i: (0, i, 0)),
            pl.BlockSpec((2, ROWB, 16), lambda i: (0, i, 0)),
            pl.BlockSpec((1, 64), lambda i: (0, 0)),
            pl.BlockSpec((1, 64), lambda i: (0, 0)),
            pl.BlockSpec((16, 64), lambda i: (0, 0)),
            pl.BlockSpec((64, 16), lambda i: (0, 0)),
            pl.BlockSpec((64, 16), lambda i: (0, 0)),
            pl.BlockSpec((1, 16), lambda i: (0, 0)),
            pl.BlockSpec((1, 16), lambda i: (0, 0)),
            pl.BlockSpec((16, 16), lambda i: (0, 0)),
            pl.BlockSpec((1, 16), lambda i: (0, 0)),
        ],
        out_specs=[
            pl.BlockSpec((ROWB, 16), lambda i: (i, 0)),
            pl.BlockSpec((ROWB, 16), lambda i: (i, 0)),
            pl.BlockSpec((ROWB, 16), lambda i: (i, 0)),
        ],
        out_shape=[
            jax.ShapeDtypeStruct((NP, 16), jnp.float32),
            jax.ShapeDtypeStruct((NP, 16), jnp.float32),
            jax.ShapeDtypeStruct((NP, 16), jnp.float32),
        ],
    )(accm, acce, blo, bhi, gta, w3l, w3h, asv3, adv3, ones16, c2row)


def _tc_fin_body(acc_ref, b_ref, e2_ref, o_ref):
    r = acc_ref[0] + acc_ref[1]
    d = jnp.dot(r, e2_ref[...], preferred_element_type=jnp.float32)
    o_ref[...] = r / (d + EPS) + b_ref[...]


def _tc_fin(acc3, b3p, e2):
    return pl.pallas_call(
        _tc_fin_body,
        grid=(GRID,),
        in_specs=[
            pl.BlockSpec((2, ROWB, 16), lambda i: (0, i, 0)),
            pl.BlockSpec((1, 16), lambda i: (0, 0)),
            pl.BlockSpec((16, 16), lambda i: (0, 0)),
        ],
        out_specs=pl.BlockSpec((ROWB, 16), lambda i: (i, 0)),
        out_shape=jax.ShapeDtypeStruct((NP, 16), jnp.float32),
    )(acc3, b3p, e2)


_MESH = plsc.VectorSubcoreMesh(core_axis_name="c", subcore_axis_name="s")
_RPT = NP // 16


def _sc_big_body(h2_hbm, sas2_hbm, sad2_hbm, src_hbm, dst_hbm, outm_hbm, oute_hbm,
                 idx_s, idx_d, hg0, hg1, sasg0, sasg1, sadg0, sadg1,
                 mb, eb, accm, acce, gsem0, gsem1):
    cid = lax.axis_index("c")
    sid = lax.axis_index("s")
    hg = (hg0, hg1)
    sasg = (sasg0, sasg1)
    sadg = (sadg0, sadg1)
    gsem = (gsem0, gsem1)

    zv = jnp.zeros((16,), jnp.float32)

    def zero_row(r, carry):
        for j in range(4):
            mb[r, pl.ds(16 * j, 16)] = zv
        eb[r, pl.ds(0, 16)] = zv
        return carry

    lax.fori_loop(0, CHUNK, zero_row, 0)
    for j in range(_RPT // CHUNK):
        pltpu.sync_copy(mb, accm.at[pl.ds(sid * _RPT + j * CHUNK, CHUNK)])
        pltpu.sync_copy(eb, acce.at[pl.ds(sid * _RPT + j * CHUNK, CHUNK)])
    plsc.subcore_barrier()

    pltpu.sync_copy(src_hbm.at[sid], idx_s)
    pltpu.sync_copy(dst_hbm.at[sid], idx_d)

    def fire_gather(cc, b):
        pltpu.async_copy(sas2_hbm.at[cid].at[idx_s.at[cc]], sasg[b], gsem[b])
        pltpu.async_copy(sad2_hbm.at[cid].at[idx_d.at[cc]], sadg[b], gsem[b])
        pltpu.async_copy(h2_hbm.at[cid].at[idx_s.at[cc]], hg[b], gsem[b])

    def drain_gather(cc, b):
        pltpu.make_async_copy(sas2_hbm.at[cid].at[idx_s.at[cc]], sasg[b], gsem[b]).wait()
        pltpu.make_async_copy(sad2_hbm.at[cid].at[idx_d.at[cc]], sadg[b], gsem[b]).wait()
        pltpu.make_async_copy(h2_hbm.at[cid].at[idx_s.at[cc]], hg[b], gsem[b]).wait()

    fire_gather(0, 0)
    fire_gather(1, 1)

    def body(i, carry):
        ci = 2 * i
        for b in range(2):
            cc = ci + b
            drain_gather(cc, b)

            @plsc.parallel_loop(0, CHUNK, unroll=4)
            def edge_body(e):
                al = sasg[b][e, pl.ds(0, 16)] + sadg[b][e, pl.ds(0, 16)]
                al = jnp.where(al >= 0, al, al * 0.2)
                exv = jnp.exp(al)
                eb[e, pl.ds(0, 16)] = exv
                for k in range(4):
                    bc = exv[jnp.full((16,), k, jnp.int32)]
                    mb[e, pl.ds(16 * k, 16)] = hg[b][e, pl.ds(16 * k, 16)] * bc

            @pl.when(cc + 2 < NCHB)
            def _():
                fire_gather(cc + 2, b)

            pltpu.sync_copy(mb, accm.at[idx_d.at[cc]], add=True)
            pltpu.sync_copy(eb, acce.at[idx_d.at[cc]], add=True)
        return carry

    lax.fori_loop(0, NCHB // 2, body, 0)
    plsc.subcore_barrier()

    for j in range(_RPT // CHUNK):
        r0 = sid * _RPT + j * CHUNK
        pltpu.sync_copy(accm.at[pl.ds(r0, CHUNK)], hg0)
        pltpu.sync_copy(hg0, outm_hbm.at[cid, pl.ds(r0, CHUNK)])
        pltpu.sync_copy(acce.at[pl.ds(r0, CHUNK)], eb)
        pltpu.sync_copy(eb, oute_hbm.at[cid, pl.ds(r0, CHUNK)])


_sc_big = functools.partial(
    pl.kernel,
    out_type=[
        jax.ShapeDtypeStruct((2, NP, 64), jnp.float32),
        jax.ShapeDtypeStruct((2, NP, 16), jnp.float32),
    ],
    mesh=_MESH,
    compiler_params=pltpu.CompilerParams(use_tc_tiling_on_sc=False),
    scratch_types=[
        pltpu.VMEM((NCHB, CHUNK), jnp.int32),
        pltpu.VMEM((NCHB, CHUNK), jnp.int32),
        pltpu.VMEM((CHUNK, 64), jnp.float32),
        pltpu.VMEM((CHUNK, 64), jnp.float32),
        pltpu.VMEM((CHUNK, 16), jnp.float32),
        pltpu.VMEM((CHUNK, 16), jnp.float32),
        pltpu.VMEM((CHUNK, 16), jnp.float32),
        pltpu.VMEM((CHUNK, 16), jnp.float32),
        pltpu.VMEM((CHUNK, 64), jnp.float32),
        pltpu.VMEM((CHUNK, 16), jnp.float32),
        pltpu.VMEM_SHARED((NP, 64), jnp.float32),
        pltpu.VMEM_SHARED((NP, 16), jnp.float32),
        pltpu.SemaphoreType.DMA,
        pltpu.SemaphoreType.DMA,
    ],
)(_sc_big_body)


def _sc_small_body(h3_hbm, sas3_hbm, sad3_hbm, src_hbm, dst_hbm, out_hbm,
                   idx_s, idx_d, hg, sasg, sadg, mb, acc,
                   sem1, sem2, sem3):
    cid = lax.axis_index("c")
    sid = lax.axis_index("s")
    wid = sid * 2 + cid

    zv = jnp.zeros((16,), jnp.float32)

    def zero_row(r, carry):
        mb[r, pl.ds(0, 16)] = zv
        return carry

    lax.fori_loop(0, CHUNK, zero_row, 0)
    for j in range(_RPT // CHUNK):
        pltpu.sync_copy(mb, acc.at[pl.ds(sid * _RPT + j * CHUNK, CHUNK)])
    plsc.subcore_barrier()

    pltpu.sync_copy(src_hbm.at[wid], idx_s)
    pltpu.sync_copy(dst_hbm.at[wid], idx_d)

    def chunk_body(ci, carry):
        pltpu.async_copy(sas3_hbm.at[idx_s.at[ci]], sasg, sem1)
        pltpu.async_copy(sad3_hbm.at[idx_d.at[ci]], sadg, sem2)
        pltpu.async_copy(h3_hbm.at[idx_s.at[ci]], hg, sem3)
        pltpu.make_async_copy(sas3_hbm.at[idx_s.at[ci]], sasg, sem1).wait()
        pltpu.make_async_copy(sad3_hbm.at[idx_d.at[ci]], sadg, sem2).wait()
        pltpu.make_async_copy(h3_hbm.at[idx_s.at[ci]], hg, sem3).wait()

        def edge_body(e, c2):
            al = sasg[e, pl.ds(0, 16)] + sadg[e, pl.ds(0, 16)]
            al = jnp.where(al >= 0, al, al * 0.2)
            exv = jnp.exp(al)
            mb[e, pl.ds(0, 16)] = hg[e, pl.ds(0, 16)] * exv
            return c2

        lax.fori_loop(0, CHUNK, edge_body, 0)
        pltpu.sync_copy(mb, acc.at[idx_d.at[ci]], add=True)
        return carry

    lax.fori_loop(0, NCHS, chunk_body, 0)
    plsc.subcore_barrier()

    for j in range(_RPT // CHUNK):
        r0 = sid * _RPT + j * CHUNK
        pltpu.sync_copy(acc.at[pl.ds(r0, CHUNK)], mb)
        pltpu.sync_copy(mb, out_hbm.at[cid, pl.ds(r0, CHUNK)])


_sc_small = functools.partial(
    pl.kernel,
    out_type=[jax.ShapeDtypeStruct((2, NP, 16), jnp.float32)],
    mesh=_MESH,
    compiler_params=pltpu.CompilerParams(use_tc_tiling_on_sc=False),
    scratch_types=[
        pltpu.VMEM((NCHS, CHUNK), jnp.int32),
        pltpu.VMEM((NCHS, CHUNK), jnp.int32),
        pltpu.VMEM((CHUNK, 16), jnp.float32),
        pltpu.VMEM((CHUNK, 16), jnp.float32),
        pltpu.VMEM((CHUNK, 16), jnp.float32),
        pltpu.VMEM((CHUNK, 16), jnp.float32),
        pltpu.VMEM_SHARED((NP, 16), jnp.float32),
        pltpu.SemaphoreType.DMA,
        pltpu.SemaphoreType.DMA,
        pltpu.SemaphoreType.DMA,
    ],
)(_sc_small_body)


def kernel(x, edge_index, W1, a_src1, a_dst1, b1, W2, a_src2, a_dst2, b2,
           W3, a_src3, a_dst3, b3):
    f32 = jnp.float32
    head = jnp.arange(128) // 16
    j16 = jnp.arange(16)
    glo = ((head[:, None] == j16[None, :] % 4) & (head[:, None] < 4)).astype(f32)
    ghi = ((head[:, None] - 4 == j16[None, :] % 4) & (head[:, None] >= 4)).astype(f32)
    h64 = jnp.arange(64) // 16
    gta = ((j16[:, None] == h64[None, :]) & (j16[:, None] < 4)).astype(f32)
    ones16 = jnp.ones((16, 16), f32)
    e2 = (j16[:, None] == 2).astype(f32) * jnp.ones((1, 16), f32)
    c2row = (j16[None, :] == 2).astype(f32)

    loop = jnp.arange(N, dtype=jnp.int32)
    pad = jnp.full((EP - E - N,), DN, jnp.int32)
    src_flat = jnp.concatenate([edge_index[0], loop, pad])
    dst_flat = jnp.concatenate([edge_index[1], loop, pad])
    src16 = src_flat.reshape(16, NCHB, CHUNK)
    dst16 = dst_flat.reshape(16, NCHB, CHUNK)
    src32 = src_flat.reshape(32, NCHS, CHUNK)
    dst32 = dst_flat.reshape(32, NCHS, CHUNK)

    xp = jnp.pad(x, ((0, NP - N), (0, 0)))
    asv1 = a_src1.reshape(1, 128)
    adv1 = a_dst1.reshape(1, 128)
    asv2 = a_src2.reshape(1, 128)
    adv2 = a_dst2.reshape(1, 128)
    w3p = jnp.pad(W3, ((0, 0), (0, 14)))
    asv3 = jnp.pad(a_src3.reshape(1, 2), ((0, 0), (0, 14)))
    adv3 = jnp.pad(a_dst3.reshape(1, 2), ((0, 0), (0, 14)))
    b1lo = b1[:64].reshape(1, 64)
    b1hi = b1[64:].reshape(1, 64)
    b2lo = b2[:64].reshape(1, 64)
    b2hi = b2[64:].reshape(1, 64)
    b3p = jnp.pad(b3.reshape(1, 2), ((0, 0), (0, 14)))

    h2a, sas, sad = _tc_first(xp, W1, asv1, adv1, glo, ghi)
    accm, acce = _sc_big(h2a, sas, sad, src16, dst16)
    h2b, sas, sad = _tc_mid(accm, acce, b1lo, b1hi, gta,
                            W2[:64], W2[64:], asv2, adv2, glo, ghi)
    accm, acce = _sc_big(h2b, sas, sad, src16, dst16)
    h3t, sas3, sad3 = _tc_mid3(accm, acce, b2lo, b2hi, gta,
                               w3p[:64], w3p[64:], asv3, adv3, ones16, c2row)
    (acc3,) = _sc_small(h3t, sas3, sad3, src32, dst32)
    o = _tc_fin(acc3, b3p, e2)
    return o[:N, :2]
